# baseline (speedup 1.0000x reference)
import functools

import jax
import jax.numpy as jnp
from jax import lax
from jax.experimental import pallas as pl
from jax.experimental.pallas import tpu as pltpu
from jax.experimental.pallas import tpu_sc as plsc

_VOCAB = 50272
_DM = 2048
_SEQ = 2048
_HASH = 10
_BITS = 16
_MR, _MC = 392, 128
_MAIN = _MR * _MC
_TAIL = _VOCAB - _MAIN
_GROWS = 16
_CHUNKS = _DM // 16

_ROT_A = (13, 15, 26, 6)
_ROT_B = (17, 29, 16, 24)
_MAGIC = 0x1BD11BDA


def _rotl(x, d):
    return lax.shift_left(x, d) | lax.shift_right_logical(x, 32 - d)


def _threefry2x32(k0, k1, x0, x1):
    ks = [k0, k1, k0 ^ k1 ^ _MAGIC]
    x0 = x0 + ks[0]
    x1 = x1 + ks[1]
    for r in range(5):
        for rot in (_ROT_A if r % 2 == 0 else _ROT_B):
            x0 = x0 + x1
            x1 = _rotl(x1, rot)
            x1 = x0 ^ x1
        x0 = x0 + ks[(r + 1) % 3]
        x1 = x1 + ks[(r + 2) % 3] + (r + 1)
    return x0, x1


def _uniform(k0, k1, pidx):
    o0, o1 = _threefry2x32(k0, k1, jnp.zeros_like(pidx), pidx)
    ub = lax.shift_right_logical(o0 ^ o1, 9) | 0x3F800000
    return jnp.maximum(lax.bitcast_convert_type(ub, jnp.float32) - 1.0, 0.0)


@functools.partial(
    pl.kernel,
    out_type=jax.ShapeDtypeStruct((_BITS, 16), jnp.float32),
    mesh=plsc.VectorSubcoreMesh(core_axis_name="c", subcore_axis_name="s"),
    scratch_types=[
        pltpu.VMEM((_GROWS,), jnp.int32),
        pltpu.VMEM((_GROWS, _DM), jnp.float32),
        pltpu.VMEM((_DM,), jnp.float32),
        pltpu.VMEM((_DM,), jnp.float32),
        pltpu.VMEM((1, 16), jnp.float32),
        pltpu.VMEM_SHARED((_DM,), jnp.float32),
        pltpu.SemaphoreType.DMA,
    ],
)
def _sc_seed(ids_hbm, rvec_hbm, embed_hbm, out_hbm,
             idx_v, rows_v, mean_v, rrow_v, pp_v, mean_sh, sem):
    cid = lax.axis_index("c")
    sid = lax.axis_index("s")

    @pl.when((cid == 0) & (sid == 0))
    def _gather_and_mean():
        pltpu.sync_copy(ids_hbm.at[pl.ds(_SEQ - _GROWS, _GROWS)], idx_v)
        pltpu.async_copy(embed_hbm.at[idx_v], rows_v, sem).wait()
        for k in range(_CHUNKS):
            sl = pl.ds(k * 16, 16)
            acc = rows_v[_GROWS - _HASH, sl]
            for j in range(_GROWS - _HASH + 1, _GROWS):
                acc = acc + rows_v[j, sl]
            mean_v[sl] = acc / jnp.float32(_HASH)
        pltpu.sync_copy(mean_v, mean_sh)

    plsc.subcore_barrier()

    @pl.when(cid == 0)
    def _dot_row():
        pltpu.sync_copy(mean_sh, mean_v)
        pltpu.sync_copy(rvec_hbm.at[sid], rrow_v)
        acc = jnp.zeros((16,), jnp.float32)
        for k in range(_CHUNKS):
            sl = pl.ds(k * 16, 16)
            acc = acc + rrow_v[sl] * mean_v[sl]
        pp_v[0, :] = acc
        pltpu.sync_copy(pp_v.at[0], out_hbm.at[sid])


def _body(pp_ref, lmain_hbm, lflat_hbm, out_hbm,
          l_ref, lf_ref, o_ref, pt_ref, lsem, osem):
    lcopy = pltpu.make_async_copy(lmain_hbm, l_ref, lsem)
    lcopy.start()
    fcopy = pltpu.make_async_copy(lflat_hbm, lf_ref, lsem)
    fcopy.start()

    o_ref[...] = jnp.full((1, _VOCAB), -100000.0, jnp.float32)
    ocopy = pltpu.make_async_copy(o_ref, out_hbm, osem)
    ocopy.start()

    lcopy.wait()
    fcopy.wait()
    l = l_ref[...]
    lt = lf_ref[0:1, pl.ds(_MAIN, _TAIL)]
    tci = lax.broadcasted_iota(jnp.int32, (1, _TAIL), 1)
    m = jnp.maximum(jnp.max(l, keepdims=True)[:1, :1],
                    jnp.max(lt, keepdims=True)[:1, :1])
    s = (jnp.sum(jnp.exp(l - m), keepdims=True)[:1, :1]
         + jnp.sum(jnp.exp(lt - m), keepdims=True)[:1, :1])
    c_const = m + jnp.log(s)

    proj = jnp.sum(pp_ref[...], axis=1, keepdims=True)
    sbits = (proj > 0).astype(jnp.int32)
    brow = lax.broadcasted_iota(jnp.int32, (_BITS, 1), 0)
    powers = lax.shift_left(jnp.int32(1), (_BITS - 1) - brow)
    seed = jnp.sum(sbits * powers, keepdims=True)[:1, :1]

    z = jnp.zeros((1, 1), jnp.int32)
    k0, k1 = _threefry2x32(z, z, z, seed)

    pmain = (lax.broadcasted_iota(jnp.int32, (_MR, _MC), 0) * _MC
             + lax.broadcasted_iota(jnp.int32, (_MR, _MC), 1))
    xmain = _uniform(k0, k1, pmain)
    ptail = _MAIN + tci
    xtail = _uniform(k0, k1, ptail)

    pos_inf = jnp.float32(jnp.inf)
    rmain = jnp.where(xmain > 0, (c_const - l) / xmain, pos_inf)
    rtail = jnp.where(xtail > 0, (c_const - lt) / xtail, pos_inf)
    rmin = jnp.minimum(jnp.min(rmain, keepdims=True)[:1, :1],
                       jnp.min(rtail, keepdims=True)[:1, :1])
    big = jnp.int32(2**30)
    widx = jnp.minimum(
        jnp.min(jnp.where(rmain == rmin, pmain, big), keepdims=True)[:1, :1],
        jnp.min(jnp.where(rtail == rmin, ptail, big), keepdims=True)[:1, :1])
    widx_s = jnp.min(jnp.where(rmain == rmin, pmain, big))
    widx_s = jnp.minimum(widx_s, jnp.min(jnp.where(rtail == rmin, ptail, big)))

    ocopy.wait()

    @pl.when(widx_s < _MAIN)
    def _patch_aligned():
        base = (widx_s // 128) * 128
        pci = lax.broadcasted_iota(jnp.int32, (1, 128), 1)
        pt_ref[...] = jnp.where(pci == (widx - base), jnp.float32(100000.0),
                                jnp.float32(-100000.0))
        pcopy = pltpu.make_async_copy(
            pt_ref, out_hbm.at[0:1, pl.ds(base, 128)], osem)
        pcopy.start()
        pcopy.wait()

    @pl.when(widx_s >= _MAIN)
    def _patch_tail():
        ci = lax.broadcasted_iota(jnp.int32, (1, _VOCAB), 1)
        o_ref[...] = jnp.where(ci == widx, jnp.float32(100000.0),
                               jnp.float32(-100000.0))
        pcopy = pltpu.make_async_copy(o_ref, out_hbm, osem)
        pcopy.start()
        pcopy.wait()


def _run(pp, lmain, lflat, interpret=False):
    return pl.pallas_call(
        _body,
        out_shape=jax.ShapeDtypeStruct((1, _VOCAB), jnp.float32),
        in_specs=[
            pl.BlockSpec(memory_space=pltpu.VMEM),
            pl.BlockSpec(memory_space=pl.ANY),
            pl.BlockSpec(memory_space=pl.ANY),
        ],
        out_specs=pl.BlockSpec(memory_space=pl.ANY),
        scratch_shapes=[
            pltpu.VMEM((_MR, _MC), jnp.float32),
            pltpu.VMEM((1, _VOCAB), jnp.float32),
            pltpu.VMEM((1, _VOCAB), jnp.float32),
            pltpu.VMEM((1, 128), jnp.float32),
            pltpu.SemaphoreType.DMA,
            pltpu.SemaphoreType.DMA,
        ],
        interpret=interpret,
    )(pp, lmain, lflat)


def kernel(input_ids, logits, embed_tokens, r_vectors):
    ids1d = input_ids.reshape(_SEQ).astype(jnp.int32)
    pp = _sc_seed(ids1d, r_vectors, embed_tokens)
    lmain = logits[0, :_MAIN].reshape(_MR, _MC)
    return _run(pp, lmain, logits)


# speedup vs baseline: 5.3060x; 5.3060x over previous
import jax
import jax.numpy as jnp
from jax import lax
from jax.experimental import pallas as pl
from jax.experimental.pallas import tpu as pltpu

_VOCAB = 50272
_DM = 2048
_SEQ = 2048
_HASH = 10
_BITS = 16
_MR, _MC = 392, 128
_MAIN = _MR * _MC
_TAIL = _VOCAB - _MAIN

_ROT_A = (13, 15, 26, 6)
_ROT_B = (17, 29, 16, 24)
_MAGIC = 0x1BD11BDA


def _rotl(x, d):
    return lax.shift_left(x, d) | lax.shift_right_logical(x, 32 - d)


def _threefry2x32(k0, k1, x0, x1):
    ks = [k0, k1, k0 ^ k1 ^ _MAGIC]
    x0 = x0 + ks[0]
    x1 = x1 + ks[1]
    for r in range(5):
        for rot in (_ROT_A if r % 2 == 0 else _ROT_B):
            x0 = x0 + x1
            x1 = _rotl(x1, rot)
            x1 = x0 ^ x1
        x0 = x0 + ks[(r + 1) % 3]
        x1 = x1 + ks[(r + 2) % 3] + (r + 1)
    return x0, x1


def _uniform(k0, k1, pidx):
    o0, o1 = _threefry2x32(k0, k1, jnp.zeros_like(pidx), pidx)
    ub = lax.shift_right_logical(o0 ^ o1, 9) | 0x3F800000
    return jnp.maximum(lax.bitcast_convert_type(ub, jnp.float32) - 1.0, 0.0)


def _body(ids_ref, lmain_hbm, lflat_hbm, embed_ref, r_ref, out_hbm,
          l_ref, lf_ref, o_ref, pt_ref, rows_ref, lsem, osem, gsem):
    gcopies = [
        pltpu.make_async_copy(
            embed_ref.at[pl.ds(ids_ref[0, _SEQ - _HASH + j], 1)],
            rows_ref.at[pl.ds(j, 1)], gsem)
        for j in range(_HASH)
    ]
    for c in gcopies:
        c.start()
    lcopy = pltpu.make_async_copy(lmain_hbm, l_ref, lsem)
    lcopy.start()
    fcopy = pltpu.make_async_copy(lflat_hbm, lf_ref, lsem)
    fcopy.start()

    o_ref[...] = jnp.full((1, _VOCAB), -100000.0, jnp.float32)
    ocopy = pltpu.make_async_copy(o_ref, out_hbm, osem)
    ocopy.start()

    lcopy.wait()
    fcopy.wait()
    l = l_ref[...]
    lt = lf_ref[0:1, pl.ds(_MAIN, _TAIL)]
    tci = lax.broadcasted_iota(jnp.int32, (1, _TAIL), 1)
    m = jnp.maximum(jnp.max(l, keepdims=True)[:1, :1],
                    jnp.max(lt, keepdims=True)[:1, :1])
    s = (jnp.sum(jnp.exp(l - m), keepdims=True)[:1, :1]
         + jnp.sum(jnp.exp(lt - m), keepdims=True)[:1, :1])
    c_const = m + jnp.log(s)

    for c in gcopies:
        c.wait()
    v = jnp.sum(rows_ref[...], axis=0, keepdims=True) / jnp.float32(_HASH)
    proj = jnp.sum(r_ref[...] * v, axis=1, keepdims=True)
    bits = (proj > 0).astype(jnp.int32)
    row = lax.broadcasted_iota(jnp.int32, (_BITS, 1), 0)
    powers = lax.shift_left(jnp.int32(1), (_BITS - 1) - row)
    seed = jnp.sum(bits * powers, keepdims=True)[:1, :1]

    z = jnp.zeros((1, 1), jnp.int32)
    k0, k1 = _threefry2x32(z, z, z, seed)

    pmain = (lax.broadcasted_iota(jnp.int32, (_MR, _MC), 0) * _MC
             + lax.broadcasted_iota(jnp.int32, (_MR, _MC), 1))
    xmain = _uniform(k0, k1, pmain)
    ptail = _MAIN + tci
    xtail = _uniform(k0, k1, ptail)

    pos_inf = jnp.float32(jnp.inf)
    rmain = jnp.where(xmain > 0, (c_const - l) / xmain, pos_inf)
    rtail = jnp.where(xtail > 0, (c_const - lt) / xtail, pos_inf)
    rmin = jnp.minimum(jnp.min(rmain, keepdims=True)[:1, :1],
                       jnp.min(rtail, keepdims=True)[:1, :1])
    big = jnp.int32(2**30)
    widx = jnp.minimum(
        jnp.min(jnp.where(rmain == rmin, pmain, big), keepdims=True)[:1, :1],
        jnp.min(jnp.where(rtail == rmin, ptail, big), keepdims=True)[:1, :1])
    widx_s = jnp.min(jnp.where(rmain == rmin, pmain, big))
    widx_s = jnp.minimum(widx_s, jnp.min(jnp.where(rtail == rmin, ptail, big)))

    ocopy.wait()

    @pl.when(widx_s < _MAIN)
    def _patch_aligned():
        base = (widx_s // 128) * 128
        pci = lax.broadcasted_iota(jnp.int32, (1, 128), 1)
        pt_ref[...] = jnp.where(pci == (widx - base), jnp.float32(100000.0),
                                jnp.float32(-100000.0))
        pcopy = pltpu.make_async_copy(
            pt_ref, out_hbm.at[0:1, pl.ds(base, 128)], osem)
        pcopy.start()
        pcopy.wait()

    @pl.when(widx_s >= _MAIN)
    def _patch_tail():
        ci = lax.broadcasted_iota(jnp.int32, (1, _VOCAB), 1)
        o_ref[...] = jnp.where(ci == widx, jnp.float32(100000.0),
                               jnp.float32(-100000.0))
        pcopy = pltpu.make_async_copy(o_ref, out_hbm, osem)
        pcopy.start()
        pcopy.wait()


def _run(ids, lmain, lflat, embed_tokens, r_vectors, interpret=False):
    return pl.pallas_call(
        _body,
        out_shape=jax.ShapeDtypeStruct((1, _VOCAB), jnp.float32),
        in_specs=[
            pl.BlockSpec(memory_space=pltpu.SMEM),
            pl.BlockSpec(memory_space=pl.ANY),
            pl.BlockSpec(memory_space=pl.ANY),
            pl.BlockSpec(memory_space=pl.ANY),
            pl.BlockSpec(memory_space=pltpu.VMEM),
        ],
        out_specs=pl.BlockSpec(memory_space=pl.ANY),
        scratch_shapes=[
            pltpu.VMEM((_MR, _MC), jnp.float32),
            pltpu.VMEM((1, _VOCAB), jnp.float32),
            pltpu.VMEM((1, _VOCAB), jnp.float32),
            pltpu.VMEM((1, 128), jnp.float32),
            pltpu.VMEM((_HASH, _DM), jnp.float32),
            pltpu.SemaphoreType.DMA,
            pltpu.SemaphoreType.DMA,
            pltpu.SemaphoreType.DMA,
        ],
        interpret=interpret,
    )(ids, lmain, lflat, embed_tokens, r_vectors)


def kernel(input_ids, logits, embed_tokens, r_vectors):
    ids = input_ids.astype(jnp.int32)
    lmain = logits[0, :_MAIN].reshape(_MR, _MC)
    return _run(ids, lmain, logits, embed_tokens, r_vectors)


# speedup vs baseline: 5.3479x; 1.0079x over previous
import jax
import jax.numpy as jnp
from jax import lax
from jax.experimental import pallas as pl
from jax.experimental.pallas import tpu as pltpu

_VOCAB = 50272
_DM = 2048
_SEQ = 2048
_HASH = 10
_BITS = 16
_MR, _MC = 392, 128
_MAIN = _MR * _MC
_TAIL = _VOCAB - _MAIN

_ROT_A = (13, 15, 26, 6)
_ROT_B = (17, 29, 16, 24)
_MAGIC = 0x1BD11BDA


def _rotl(x, d):
    return lax.shift_left(x, d) | lax.shift_right_logical(x, 32 - d)


def _threefry2x32(k0, k1, x0, x1):
    ks = [k0, k1, k0 ^ k1 ^ _MAGIC]
    x0 = x0 + ks[0]
    x1 = x1 + ks[1]
    for r in range(5):
        for rot in (_ROT_A if r % 2 == 0 else _ROT_B):
            x0 = x0 + x1
            x1 = _rotl(x1, rot)
            x1 = x0 ^ x1
        x0 = x0 + ks[(r + 1) % 3]
        x1 = x1 + ks[(r + 2) % 3] + (r + 1)
    return x0, x1


def _uniform(k0, k1, pidx):
    o0, o1 = _threefry2x32(k0, k1, jnp.zeros_like(pidx), pidx)
    ub = lax.shift_right_logical(o0 ^ o1, 9) | 0x3F800000
    return jnp.maximum(lax.bitcast_convert_type(ub, jnp.float32) - 1.0, 0.0)


def _body(ids_ref, lmain_hbm, lflat_hbm, embed_ref, r_ref, out_hbm,
          l_ref, lf_ref, o_ref, pt_ref, rows_ref, lsem, osem, gsem):
    gcopies = [
        pltpu.make_async_copy(
            embed_ref.at[pl.ds(ids_ref[0, _SEQ - _HASH + j], 1)],
            rows_ref.at[pl.ds(j, 1)], gsem)
        for j in range(_HASH)
    ]
    for c in gcopies:
        c.start()
    lcopy = pltpu.make_async_copy(lmain_hbm, l_ref, lsem)
    lcopy.start()
    fcopy = pltpu.make_async_copy(lflat_hbm, lf_ref, lsem)
    fcopy.start()

    o_ref[...] = jnp.full((1, _VOCAB), -100000.0, jnp.float32)
    ocopy = pltpu.make_async_copy(o_ref, out_hbm, osem)
    ocopy.start()

    lcopy.wait()
    fcopy.wait()
    l = l_ref[...]
    lt = lf_ref[0:1, pl.ds(_MAIN, _TAIL)]
    tci = lax.broadcasted_iota(jnp.int32, (1, _TAIL), 1)
    m = jnp.maximum(jnp.max(l, keepdims=True)[:1, :1],
                    jnp.max(lt, keepdims=True)[:1, :1])
    s = (jnp.sum(jnp.exp(l - m), keepdims=True)[:1, :1]
         + jnp.sum(jnp.exp(lt - m), keepdims=True)[:1, :1])
    c_const = m + jnp.log(s)

    for c in gcopies:
        c.wait()
    v = jnp.sum(rows_ref[...], axis=0, keepdims=True) / jnp.float32(_HASH)
    proj = jnp.sum(r_ref[...] * v, axis=1, keepdims=True)
    bits = (proj > 0).astype(jnp.int32)
    row = lax.broadcasted_iota(jnp.int32, (_BITS, 1), 0)
    powers = lax.shift_left(jnp.int32(1), (_BITS - 1) - row)
    seed = jnp.sum(bits * powers, keepdims=True)[:1, :1]

    z = jnp.zeros((1, 1), jnp.int32)
    k0, k1 = _threefry2x32(z, z, z, seed)

    pmain = (lax.broadcasted_iota(jnp.int32, (_MR, _MC), 0) * _MC
             + lax.broadcasted_iota(jnp.int32, (_MR, _MC), 1))
    xmain = _uniform(k0, k1, pmain)
    ptail = _MAIN + tci
    xtail = _uniform(k0, k1, ptail)

    pos_inf = jnp.float32(jnp.inf)
    rmain = jnp.where(xmain > 0, (c_const - l) / xmain, pos_inf)
    rtail = jnp.where(xtail > 0, (c_const - lt) / xtail, pos_inf)
    rmin = jnp.minimum(jnp.min(rmain), jnp.min(rtail))
    big = jnp.int32(2**30)
    widx = jnp.minimum(
        jnp.min(jnp.where(rmain == rmin, pmain, big)),
        jnp.min(jnp.where(rtail == rmin, ptail, big)))

    ocopy.wait()

    @pl.when(widx < _MAIN)
    def _patch_aligned():
        base = (widx // 128) * 128
        pci = lax.broadcasted_iota(jnp.int32, (1, 128), 1)
        pt_ref[...] = jnp.where(pci == (widx - base), jnp.float32(100000.0),
                                jnp.float32(-100000.0))
        pcopy = pltpu.make_async_copy(
            pt_ref, out_hbm.at[0:1, pl.ds(base, 128)], osem)
        pcopy.start()
        pcopy.wait()

    @pl.when(widx >= _MAIN)
    def _patch_tail():
        ci = lax.broadcasted_iota(jnp.int32, (1, _VOCAB), 1)
        o_ref[...] = jnp.where(ci == widx, jnp.float32(100000.0),
                               jnp.float32(-100000.0))
        pcopy = pltpu.make_async_copy(o_ref, out_hbm, osem)
        pcopy.start()
        pcopy.wait()


def _run(ids, lmain, lflat, embed_tokens, r_vectors, interpret=False):
    return pl.pallas_call(
        _body,
        out_shape=jax.ShapeDtypeStruct((1, _VOCAB), jnp.float32),
        in_specs=[
            pl.BlockSpec(memory_space=pltpu.SMEM),
            pl.BlockSpec(memory_space=pl.ANY),
            pl.BlockSpec(memory_space=pl.ANY),
            pl.BlockSpec(memory_space=pl.ANY),
            pl.BlockSpec(memory_space=pltpu.VMEM),
        ],
        out_specs=pl.BlockSpec(memory_space=pl.ANY),
        scratch_shapes=[
            pltpu.VMEM((_MR, _MC), jnp.float32),
            pltpu.VMEM((1, _VOCAB), jnp.float32),
            pltpu.VMEM((1, _VOCAB), jnp.float32),
            pltpu.VMEM((1, 128), jnp.float32),
            pltpu.VMEM((_HASH, _DM), jnp.float32),
            pltpu.SemaphoreType.DMA,
            pltpu.SemaphoreType.DMA,
            pltpu.SemaphoreType.DMA,
        ],
        interpret=interpret,
    )(ids, lmain, lflat, embed_tokens, r_vectors)


def kernel(input_ids, logits, embed_tokens, r_vectors):
    ids = input_ids.astype(jnp.int32)
    lmain = logits[0, :_MAIN].reshape(_MR, _MC)
    return _run(ids, lmain, logits, embed_tokens, r_vectors)
